# Initial kernel scaffold; baseline (speedup 1.0000x reference)
import jax
import jax.numpy as jnp
from jax.experimental import pallas as pl


def kernel(x, edge_index, W1, b1, W2, b2, W3, b3):
    raise NotImplementedError("write your pallas kernel here")


# speedup vs baseline: 20.4119x; 20.4119x over previous
import jax
import jax.numpy as jnp
from jax import lax
from jax.experimental import pallas as pl
from jax.experimental.pallas import tpu as pltpu
from jax.experimental.pallas import tpu_sc as plsc

N = 10000
NP = 10240
E = 320000
D = 64
DW = 16
K = 80
NC = 2
NS = 16
NW = NC * NS
CPW = E // (NW * K)
RPT = NP // NS
RB = 160
LANES = 16


def _zero_vmem(ref, rows, width):
    @pl.loop(0, rows)
    def _(i):
        for k in range(width // LANES):
            ref[i, pl.ds(k * LANES, LANES)] = jnp.zeros((LANES,), jnp.float32)


def _sc_scatter_body(table, row_r, col_r, out, accum, rows_v, cols_v, buf,
                     zbuf, sem):
    cid = lax.axis_index("c")
    sid = lax.axis_index("s")
    wid = sid * NC + cid

    _zero_vmem(zbuf, RB, D)
    for k in range(RPT // RB):
        pltpu.sync_copy(zbuf, accum.at[pl.ds(sid * RPT + k * RB, RB)])
    pltpu.sync_copy(row_r.at[wid], rows_v)
    pltpu.sync_copy(col_r.at[wid], cols_v)
    plsc.subcore_barrier()

    @pl.loop(0, CPW)
    def _(j):
        pltpu.async_copy(table.at[rows_v.at[j]], buf, sem).wait()
        pltpu.sync_copy(buf, accum.at[cols_v.at[j]], add=True)

    plsc.subcore_barrier()
    for k in range(RPT // RB):
        start = sid * RPT + k * RB
        pltpu.sync_copy(accum.at[pl.ds(start, RB)], zbuf)
        pltpu.sync_copy(zbuf, out.at[cid].at[pl.ds(start, RB)])


def _sc_scatter(table, row_r, col_r):
    mesh = plsc.VectorSubcoreMesh(core_axis_name="c", subcore_axis_name="s")
    return pl.kernel(
        _sc_scatter_body,
        out_type=jax.ShapeDtypeStruct((NC, NP, D), jnp.float32),
        mesh=mesh,
        scratch_types=[
            pltpu.VMEM_SHARED((NP, D), jnp.float32),
            pltpu.VMEM((CPW, K), jnp.int32),
            pltpu.VMEM((CPW, K), jnp.int32),
            pltpu.VMEM((K, D), jnp.float32),
            pltpu.VMEM((RB, D), jnp.float32),
            pltpu.SemaphoreType.DMA,
        ],
        compiler_params=pltpu.CompilerParams(use_tc_tiling_on_sc=False),
        name="gcn_edge_scatter",
    )(table, row_r, col_r)


def _sc_degree_body(col_r, out, accum, cols_v, ones_v, zbuf):
    cid = lax.axis_index("c")
    sid = lax.axis_index("s")
    wid = sid * NC + cid

    _zero_vmem(zbuf, RB, DW)
    for k in range(RPT // RB):
        pltpu.sync_copy(zbuf, accum.at[pl.ds(sid * RPT + k * RB, RB)])

    @pl.loop(0, K)
    def _(i):
        ones_v[i, pl.ds(0, LANES)] = jnp.ones((LANES,), jnp.float32)

    pltpu.sync_copy(col_r.at[wid], cols_v)
    plsc.subcore_barrier()

    @pl.loop(0, CPW)
    def _(j):
        pltpu.sync_copy(ones_v, accum.at[cols_v.at[j]], add=True)

    plsc.subcore_barrier()
    for k in range(RPT // RB):
        start = sid * RPT + k * RB
        pltpu.sync_copy(accum.at[pl.ds(start, RB)], zbuf)
        pltpu.sync_copy(zbuf, out.at[cid].at[pl.ds(start, RB)])


def _sc_degree(col_r):
    mesh = plsc.VectorSubcoreMesh(core_axis_name="c", subcore_axis_name="s")
    return pl.kernel(
        _sc_degree_body,
        out_type=jax.ShapeDtypeStruct((NC, NP, DW), jnp.float32),
        mesh=mesh,
        scratch_types=[
            pltpu.VMEM_SHARED((NP, DW), jnp.float32),
            pltpu.VMEM((CPW, K), jnp.int32),
            pltpu.VMEM((K, DW), jnp.float32),
            pltpu.VMEM((RB, DW), jnp.float32),
        ],
        compiler_params=pltpu.CompilerParams(use_tc_tiling_on_sc=False),
        name="gcn_degree",
    )(col_r)


def _dinv(deg_ref):
    deg = deg_ref[0] + deg_ref[1]
    return lax.rsqrt(deg[:, 0:1] + 1.0)


def _tc1_body(deg_ref, x_ref, w_ref, o_ref):
    xw = jnp.dot(x_ref[...], w_ref[...], preferred_element_type=jnp.float32)
    o_ref[...] = xw * _dinv(deg_ref)


def _tc2_body(deg_ref, s_ref, xs_ref, b_ref, w_ref, h_ref, o_ref):
    dinv = _dinv(deg_ref)
    s = s_ref[0] + s_ref[1] + xs_ref[...]
    h = jnp.maximum(s * dinv + b_ref[...], 0.0)
    h_ref[...] = h
    o_ref[...] = jnp.dot(h, w_ref[...],
                         preferred_element_type=jnp.float32) * dinv


def _tc3_body(deg_ref, s_ref, xs_ref, b_ref, h_ref, o_ref):
    dinv = _dinv(deg_ref)
    s = s_ref[0] + s_ref[1] + xs_ref[...]
    y = jnp.maximum(s * dinv + b_ref[...], 0.0)
    o_ref[...] = (y + h_ref[...]) * dinv


def _tc4_body(deg_ref, s_ref, hs_ref, w_ref, b_ref, o_ref):
    dinv = _dinv(deg_ref)
    t = s_ref[0] + s_ref[1] + hs_ref[...]
    z = jnp.dot(t, w_ref[...], preferred_element_type=jnp.float32)
    z = z * dinv + b_ref[...]
    m = jnp.max(z, axis=1, keepdims=True)
    e = z - m
    o_ref[...] = e - jnp.log(jnp.sum(jnp.exp(e), axis=1, keepdims=True))


def _tc(body, out_shape, *args):
    return pl.pallas_call(body, out_shape=out_shape)(*args)


def kernel(x, edge_index, W1, b1, W2, b2, W3, b3):
    row_r = edge_index[0].reshape(NW, CPW, K)
    col_r = edge_index[1].reshape(NW, CPW, K)
    xp = jnp.pad(x, ((0, NP - N), (0, 0)))
    f32 = jnp.float32

    deg_p = _sc_degree(col_r)
    xs1 = _tc(_tc1_body, jax.ShapeDtypeStruct((NP, D), f32), deg_p, xp, W1)
    s1p = _sc_scatter(xs1, row_r, col_r)
    h, xs2 = _tc(
        _tc2_body,
        (jax.ShapeDtypeStruct((NP, D), f32),
         jax.ShapeDtypeStruct((NP, D), f32)),
        deg_p, s1p, xs1, b1.reshape(1, D), W2)
    s2p = _sc_scatter(xs2, row_r, col_r)
    hs3 = _tc(_tc3_body, jax.ShapeDtypeStruct((NP, D), f32),
              deg_p, s2p, xs2, b2.reshape(1, D), h)
    s3p = _sc_scatter(hs3, row_r, col_r)
    out = _tc(_tc4_body, jax.ShapeDtypeStruct((NP, 10), f32),
              deg_p, s3p, hs3, W3, b3.reshape(1, 10))
    return out[:N]


# speedup vs baseline: 32.5483x; 1.5946x over previous
import jax
import jax.numpy as jnp
from jax import lax
from jax.experimental import pallas as pl
from jax.experimental.pallas import tpu as pltpu
from jax.experimental.pallas import tpu_sc as plsc

N = 10000
NP = 10240
E = 320000
D = 64
DW = 16
K = 100
NC = 2
NS = 16
NW = NC * NS
CPW = E // (NW * K)
RPT = NP // NS
RB = 160
LANES = 16


def _zero_vmem(ref, rows, width):
    @pl.loop(0, rows)
    def _(i):
        for k in range(width // LANES):
            ref[i, pl.ds(k * LANES, LANES)] = jnp.zeros((LANES,), jnp.float32)


def _sc_scatter_body(table, row_r, col_r, out, accum, rows_v, cols_v, buf_a,
                     buf_b, zbuf, sem_a, sem_b):
    cid = lax.axis_index("c")
    sid = lax.axis_index("s")
    wid = sid * NC + cid

    _zero_vmem(zbuf, RB, D)
    for k in range(RPT // RB):
        pltpu.sync_copy(zbuf, accum.at[pl.ds(sid * RPT + k * RB, RB)])
    pltpu.sync_copy(row_r.at[wid], rows_v)
    pltpu.sync_copy(col_r.at[wid], cols_v)
    plsc.subcore_barrier()

    def fire(j, buf, sem):
        pltpu.async_copy(table.at[rows_v.at[j]], buf, sem)

    def drain(buf, sem):
        pltpu.make_async_copy(table.at[pl.ds(0, K)], buf, sem).wait()

    fire(0, buf_a, sem_a)

    @pl.loop(0, CPW // 2)
    def _(p):
        fire(2 * p + 1, buf_b, sem_b)
        drain(buf_a, sem_a)
        pltpu.sync_copy(buf_a, accum.at[cols_v.at[2 * p]], add=True)

        @pl.when(p < CPW // 2 - 1)
        def _():
            fire(2 * p + 2, buf_a, sem_a)

        drain(buf_b, sem_b)
        pltpu.sync_copy(buf_b, accum.at[cols_v.at[2 * p + 1]], add=True)

    plsc.subcore_barrier()
    for k in range(RPT // RB):
        start = sid * RPT + k * RB
        pltpu.sync_copy(accum.at[pl.ds(start, RB)], zbuf)
        pltpu.sync_copy(zbuf, out.at[cid].at[pl.ds(start, RB)])


def _sc_scatter(table, row_r, col_r):
    mesh = plsc.VectorSubcoreMesh(core_axis_name="c", subcore_axis_name="s")
    return pl.kernel(
        _sc_scatter_body,
        out_type=jax.ShapeDtypeStruct((NC, NP, D), jnp.float32),
        mesh=mesh,
        scratch_types=[
            pltpu.VMEM_SHARED((NP, D), jnp.float32),
            pltpu.VMEM((CPW, K), jnp.int32),
            pltpu.VMEM((CPW, K), jnp.int32),
            pltpu.VMEM((K, D), jnp.float32),
            pltpu.VMEM((K, D), jnp.float32),
            pltpu.VMEM((RB, D), jnp.float32),
            pltpu.SemaphoreType.DMA,
            pltpu.SemaphoreType.DMA,
        ],
        compiler_params=pltpu.CompilerParams(use_tc_tiling_on_sc=False),
        name="gcn_edge_scatter",
    )(table, row_r, col_r)


def _sc_degree_body(col_r, out, accum, cols_v, ones_v, zbuf, sem):
    cid = lax.axis_index("c")
    sid = lax.axis_index("s")
    wid = sid * NC + cid

    _zero_vmem(zbuf, RB, DW)
    for k in range(RPT // RB):
        pltpu.sync_copy(zbuf, accum.at[pl.ds(sid * RPT + k * RB, RB)])

    @pl.loop(0, K)
    def _(i):
        ones_v[i, pl.ds(0, LANES)] = jnp.ones((LANES,), jnp.float32)

    pltpu.sync_copy(col_r.at[wid], cols_v)
    plsc.subcore_barrier()

    FK = 10

    @pl.loop(0, CPW // FK)
    def _(g):
        for i in range(FK):
            pltpu.async_copy(ones_v, accum.at[cols_v.at[g * FK + i]], sem,
                             add=True)
        for _i in range(FK):
            pltpu.make_async_copy(ones_v, accum.at[pl.ds(0, K)], sem).wait()

    plsc.subcore_barrier()
    for k in range(RPT // RB):
        start = sid * RPT + k * RB
        pltpu.sync_copy(accum.at[pl.ds(start, RB)], zbuf)
        pltpu.sync_copy(zbuf, out.at[cid].at[pl.ds(start, RB)])


def _sc_degree(col_r):
    mesh = plsc.VectorSubcoreMesh(core_axis_name="c", subcore_axis_name="s")
    return pl.kernel(
        _sc_degree_body,
        out_type=jax.ShapeDtypeStruct((NC, NP, DW), jnp.float32),
        mesh=mesh,
        scratch_types=[
            pltpu.VMEM_SHARED((NP, DW), jnp.float32),
            pltpu.VMEM((CPW, K), jnp.int32),
            pltpu.VMEM((K, DW), jnp.float32),
            pltpu.VMEM((RB, DW), jnp.float32),
            pltpu.SemaphoreType.DMA,
        ],
        compiler_params=pltpu.CompilerParams(use_tc_tiling_on_sc=False),
        name="gcn_degree",
    )(col_r)


def _dinv(deg_ref):
    deg = deg_ref[0] + deg_ref[1]
    return lax.rsqrt(deg[:, 0:1] + 1.0)


def _tc1_body(deg_ref, x_ref, w_ref, o_ref):
    xw = jnp.dot(x_ref[...], w_ref[...], preferred_element_type=jnp.float32)
    o_ref[...] = xw * _dinv(deg_ref)


def _tc2_body(deg_ref, s_ref, xs_ref, b_ref, w_ref, h_ref, o_ref):
    dinv = _dinv(deg_ref)
    s = s_ref[0] + s_ref[1] + xs_ref[...]
    h = jnp.maximum(s * dinv + b_ref[...], 0.0)
    h_ref[...] = h
    o_ref[...] = jnp.dot(h, w_ref[...],
                         preferred_element_type=jnp.float32) * dinv


def _tc3_body(deg_ref, s_ref, xs_ref, b_ref, h_ref, o_ref):
    dinv = _dinv(deg_ref)
    s = s_ref[0] + s_ref[1] + xs_ref[...]
    y = jnp.maximum(s * dinv + b_ref[...], 0.0)
    o_ref[...] = (y + h_ref[...]) * dinv


def _tc4_body(deg_ref, s_ref, hs_ref, w_ref, b_ref, o_ref):
    dinv = _dinv(deg_ref)
    t = s_ref[0] + s_ref[1] + hs_ref[...]
    z = jnp.dot(t, w_ref[...], preferred_element_type=jnp.float32)
    z = z * dinv + b_ref[...]
    m = jnp.max(z, axis=1, keepdims=True)
    e = z - m
    o_ref[...] = e - jnp.log(jnp.sum(jnp.exp(e), axis=1, keepdims=True))


def _tc(body, out_shape, *args):
    return pl.pallas_call(body, out_shape=out_shape)(*args)


def kernel(x, edge_index, W1, b1, W2, b2, W3, b3):
    row_r = edge_index[0].reshape(NW, CPW, K)
    col_r = edge_index[1].reshape(NW, CPW, K)
    xp = jnp.pad(x, ((0, NP - N), (0, 0)))
    f32 = jnp.float32

    deg_p = _sc_degree(col_r)
    xs1 = _tc(_tc1_body, jax.ShapeDtypeStruct((NP, D), f32), deg_p, xp, W1)
    s1p = _sc_scatter(xs1, row_r, col_r)
    h, xs2 = _tc(
        _tc2_body,
        (jax.ShapeDtypeStruct((NP, D), f32),
         jax.ShapeDtypeStruct((NP, D), f32)),
        deg_p, s1p, xs1, b1.reshape(1, D), W2)
    s2p = _sc_scatter(xs2, row_r, col_r)
    hs3 = _tc(_tc3_body, jax.ShapeDtypeStruct((NP, D), f32),
              deg_p, s2p, xs2, b2.reshape(1, D), h)
    s3p = _sc_scatter(hs3, row_r, col_r)
    out = _tc(_tc4_body, jax.ShapeDtypeStruct((NP, 10), f32),
              deg_p, s3p, hs3, W3, b3.reshape(1, 10))
    return out[:N]


# speedup vs baseline: 36.2350x; 1.1133x over previous
import jax
import jax.numpy as jnp
from jax import lax
from jax.experimental import pallas as pl
from jax.experimental.pallas import tpu as pltpu
from jax.experimental.pallas import tpu_sc as plsc

N = 10000
NP = 10240
E = 320000
D = 64
DW = 16
K = 100
NC = 2
NS = 16
NW = NC * NS
CPW = E // (NW * K)
RPT = NP // NS
RB = 160
LANES = 16


def _zero_vmem(ref, rows, width):
    @pl.loop(0, rows)
    def _(i):
        for k in range(width // LANES):
            ref[i, pl.ds(k * LANES, LANES)] = jnp.zeros((LANES,), jnp.float32)


NB = 4


def _sc_scatter_body(table, row_r, col_r, out, accum, rows_v, cols_v,
                     buf_0, buf_1, buf_2, buf_3, zbuf,
                     gsem_0, gsem_1, gsem_2, gsem_3,
                     ssem_0, ssem_1, ssem_2, ssem_3):
    bufs = (buf_0, buf_1, buf_2, buf_3)
    gsem = (gsem_0, gsem_1, gsem_2, gsem_3)
    ssem = (ssem_0, ssem_1, ssem_2, ssem_3)
    cid = lax.axis_index("c")
    sid = lax.axis_index("s")
    wid = sid * NC + cid

    _zero_vmem(zbuf, RB, D)
    for k in range(RPT // RB):
        pltpu.sync_copy(zbuf, accum.at[pl.ds(sid * RPT + k * RB, RB)])
    pltpu.sync_copy(row_r.at[wid], rows_v)
    pltpu.sync_copy(col_r.at[wid], cols_v)
    plsc.subcore_barrier()

    def fire_g(j, b):
        pltpu.async_copy(table.at[rows_v.at[j]], bufs[b], gsem[b])

    def drain_g(b):
        pltpu.make_async_copy(table.at[pl.ds(0, K)], bufs[b], gsem[b]).wait()

    def fire_s(j, b):
        pltpu.async_copy(bufs[b], accum.at[cols_v.at[j]], ssem[b], add=True)

    def drain_s(b):
        pltpu.make_async_copy(bufs[b], accum.at[pl.ds(0, K)], ssem[b]).wait()

    for b in range(NB):
        fire_g(b, b)

    G = CPW // NB

    @pl.loop(0, G)
    def _(g):
        for b in range(NB):
            drain_g(b)
            fire_s(g * NB + b, b)
        for b in range(NB):
            @pl.when(g < G - 1)
            def _():
                drain_s(b)
                fire_g((g + 1) * NB + b, b)

    for b in range(NB):
        drain_s(b)

    plsc.subcore_barrier()
    for k in range(RPT // RB):
        start = sid * RPT + k * RB
        pltpu.sync_copy(accum.at[pl.ds(start, RB)], zbuf)
        pltpu.sync_copy(zbuf, out.at[cid].at[pl.ds(start, RB)])


def _sc_scatter(table, row_r, col_r):
    mesh = plsc.VectorSubcoreMesh(core_axis_name="c", subcore_axis_name="s")
    return pl.kernel(
        _sc_scatter_body,
        out_type=jax.ShapeDtypeStruct((NC, NP, D), jnp.float32),
        mesh=mesh,
        scratch_types=[
            pltpu.VMEM_SHARED((NP, D), jnp.float32),
            pltpu.VMEM((CPW, K), jnp.int32),
            pltpu.VMEM((CPW, K), jnp.int32),
            pltpu.VMEM((K, D), jnp.float32),
            pltpu.VMEM((K, D), jnp.float32),
            pltpu.VMEM((K, D), jnp.float32),
            pltpu.VMEM((K, D), jnp.float32),
            pltpu.VMEM((RB, D), jnp.float32),
            pltpu.SemaphoreType.DMA,
            pltpu.SemaphoreType.DMA,
            pltpu.SemaphoreType.DMA,
            pltpu.SemaphoreType.DMA,
            pltpu.SemaphoreType.DMA,
            pltpu.SemaphoreType.DMA,
            pltpu.SemaphoreType.DMA,
            pltpu.SemaphoreType.DMA,
        ],
        compiler_params=pltpu.CompilerParams(use_tc_tiling_on_sc=False),
        name="gcn_edge_scatter",
    )(table, row_r, col_r)


def _sc_degree_body(col_r, out, accum, cols_v, ones_v, zbuf, sem):
    cid = lax.axis_index("c")
    sid = lax.axis_index("s")
    wid = sid * NC + cid

    _zero_vmem(zbuf, RB, DW)
    for k in range(RPT // RB):
        pltpu.sync_copy(zbuf, accum.at[pl.ds(sid * RPT + k * RB, RB)])

    @pl.loop(0, K)
    def _(i):
        ones_v[i, pl.ds(0, LANES)] = jnp.ones((LANES,), jnp.float32)

    pltpu.sync_copy(col_r.at[wid], cols_v)
    plsc.subcore_barrier()

    FK = 10

    @pl.loop(0, CPW // FK)
    def _(g):
        for i in range(FK):
            pltpu.async_copy(ones_v, accum.at[cols_v.at[g * FK + i]], sem,
                             add=True)
        for _i in range(FK):
            pltpu.make_async_copy(ones_v, accum.at[pl.ds(0, K)], sem).wait()

    plsc.subcore_barrier()
    for k in range(RPT // RB):
        start = sid * RPT + k * RB
        pltpu.sync_copy(accum.at[pl.ds(start, RB)], zbuf)
        pltpu.sync_copy(zbuf, out.at[cid].at[pl.ds(start, RB)])


def _sc_degree(col_r):
    mesh = plsc.VectorSubcoreMesh(core_axis_name="c", subcore_axis_name="s")
    return pl.kernel(
        _sc_degree_body,
        out_type=jax.ShapeDtypeStruct((NC, NP, DW), jnp.float32),
        mesh=mesh,
        scratch_types=[
            pltpu.VMEM_SHARED((NP, DW), jnp.float32),
            pltpu.VMEM((CPW, K), jnp.int32),
            pltpu.VMEM((K, DW), jnp.float32),
            pltpu.VMEM((RB, DW), jnp.float32),
            pltpu.SemaphoreType.DMA,
        ],
        compiler_params=pltpu.CompilerParams(use_tc_tiling_on_sc=False),
        name="gcn_degree",
    )(col_r)


def _dinv(deg_ref):
    deg = deg_ref[0] + deg_ref[1]
    return lax.rsqrt(deg[:, 0:1] + 1.0)


def _tc1_body(deg_ref, x_ref, w_ref, o_ref):
    xw = jnp.dot(x_ref[...], w_ref[...], preferred_element_type=jnp.float32)
    o_ref[...] = xw * _dinv(deg_ref)


def _tc2_body(deg_ref, s_ref, xs_ref, b_ref, w_ref, h_ref, o_ref):
    dinv = _dinv(deg_ref)
    s = s_ref[0] + s_ref[1] + xs_ref[...]
    h = jnp.maximum(s * dinv + b_ref[...], 0.0)
    h_ref[...] = h
    o_ref[...] = jnp.dot(h, w_ref[...],
                         preferred_element_type=jnp.float32) * dinv


def _tc3_body(deg_ref, s_ref, xs_ref, b_ref, h_ref, o_ref):
    dinv = _dinv(deg_ref)
    s = s_ref[0] + s_ref[1] + xs_ref[...]
    y = jnp.maximum(s * dinv + b_ref[...], 0.0)
    o_ref[...] = (y + h_ref[...]) * dinv


def _tc4_body(deg_ref, s_ref, hs_ref, w_ref, b_ref, o_ref):
    dinv = _dinv(deg_ref)
    t = s_ref[0] + s_ref[1] + hs_ref[...]
    z = jnp.dot(t, w_ref[...], preferred_element_type=jnp.float32)
    z = z * dinv + b_ref[...]
    m = jnp.max(z, axis=1, keepdims=True)
    e = z - m
    o_ref[...] = e - jnp.log(jnp.sum(jnp.exp(e), axis=1, keepdims=True))


def _tc(body, out_shape, *args):
    return pl.pallas_call(body, out_shape=out_shape)(*args)


def kernel(x, edge_index, W1, b1, W2, b2, W3, b3):
    row_r = edge_index[0].reshape(NW, CPW, K)
    col_r = edge_index[1].reshape(NW, CPW, K)
    xp = jnp.pad(x, ((0, NP - N), (0, 0)))
    f32 = jnp.float32

    deg_p = _sc_degree(col_r)
    xs1 = _tc(_tc1_body, jax.ShapeDtypeStruct((NP, D), f32), deg_p, xp, W1)
    s1p = _sc_scatter(xs1, row_r, col_r)
    h, xs2 = _tc(
        _tc2_body,
        (jax.ShapeDtypeStruct((NP, D), f32),
         jax.ShapeDtypeStruct((NP, D), f32)),
        deg_p, s1p, xs1, b1.reshape(1, D), W2)
    s2p = _sc_scatter(xs2, row_r, col_r)
    hs3 = _tc(_tc3_body, jax.ShapeDtypeStruct((NP, D), f32),
              deg_p, s2p, xs2, b2.reshape(1, D), h)
    s3p = _sc_scatter(hs3, row_r, col_r)
    out = _tc(_tc4_body, jax.ShapeDtypeStruct((NP, 10), f32),
              deg_p, s3p, hs3, W3, b3.reshape(1, 10))
    return out[:N]
